# baseline (speedup 1.0000x reference)
import jax
import jax.numpy as jnp
from jax.experimental import pallas as pl

B, N, S, D = 8, 512, 256, 64
DB = 8


def _tc_body(tt_ref, st_ref, out_ref):
    tt = tt_ref[...]
    st = st_ref[...]
    out_ref[0] = st[:, :, None] + tt[:, None, :]


def kernel(input_ids, time_table, space_table):
    del input_ids
    tt = time_table[:S].T
    st = space_table.T
    return pl.pallas_call(
        _tc_body,
        grid=(B, D // DB),
        in_specs=[
            pl.BlockSpec((DB, S), lambda b, j: (j, 0)),
            pl.BlockSpec((DB, N), lambda b, j: (j, 0)),
        ],
        out_specs=pl.BlockSpec((1, DB, N, S), lambda b, j: (b, j, 0, 0)),
        out_shape=jax.ShapeDtypeStruct((B, D, N, S), jnp.float32),
    )(tt, st)


# speedup vs baseline: 1.0227x; 1.0227x over previous
import jax
import jax.numpy as jnp
from jax.experimental import pallas as pl
from jax.experimental.pallas import tpu as pltpu

B, N, S, D = 8, 512, 256, 64
DB = 16


def _tc_body(t_ref, s_ref, out_ref, tt_s, st_s):
    b = pl.program_id(0)
    j = pl.program_id(1)

    @pl.when(jnp.logical_and(b == 0, j == 0))
    def _():
        tt_s[...] = t_ref[...].T
        st_s[...] = s_ref[...].T

    tt = tt_s[pl.ds(j * DB, DB), :]
    st = st_s[pl.ds(j * DB, DB), :]
    out_ref[0] = st[:, :, None] + tt[:, None, :]


def kernel(input_ids, time_table, space_table):
    del input_ids
    return pl.pallas_call(
        _tc_body,
        grid=(B, D // DB),
        in_specs=[
            pl.BlockSpec((S, D), lambda b, j: (0, 0)),
            pl.BlockSpec((N, D), lambda b, j: (0, 0)),
        ],
        out_specs=pl.BlockSpec((1, DB, N, S), lambda b, j: (b, j, 0, 0)),
        out_shape=jax.ShapeDtypeStruct((B, D, N, S), jnp.float32),
        scratch_shapes=[
            pltpu.VMEM((D, S), jnp.float32),
            pltpu.VMEM((D, N), jnp.float32),
        ],
    )(time_table, space_table)
